# baseline (speedup 1.0000x reference)
import math

import jax
import jax.numpy as jnp
from jax.experimental import pallas as pl
from jax.experimental.pallas import tpu as pltpu

B, S, D = 1, 2048, 768
H, E, TOPK, P = 12, 8, 2, 64
TS = 256
NT = S // TS
HP = H * P

_SCALE = 1.0 / math.sqrt(P)
_S = math.sqrt(_SCALE)

_HI = jax.lax.Precision.HIGHEST


def _top2_gates(logits, rs_over):
    probs = [jax.nn.sigmoid(logits[:, e * H:(e + 1) * H]) for e in range(E)]
    m1 = probs[0]
    for e in range(1, E):
        m1 = jnp.maximum(m1, probs[e])
    i1 = jnp.full(probs[0].shape, E, dtype=jnp.int32)
    for e in range(E - 1, -1, -1):
        i1 = jnp.where(probs[e] == m1, e, i1)
    neg = jnp.float32(-jnp.inf)
    q = [jnp.where(i1 == e, neg, probs[e]) for e in range(E)]
    m2 = q[0]
    for e in range(1, E):
        m2 = jnp.maximum(m2, q[e])
    i2 = jnp.full(probs[0].shape, E, dtype=jnp.int32)
    for e in range(E - 1, -1, -1):
        i2 = jnp.where(q[e] == m2, e, i2)
    denom = jnp.maximum(m1 + m2, jnp.float32(1e-9))
    scale = rs_over / denom
    gates = []
    for e in range(E):
        sel = jnp.logical_or(i1 == e, i2 == e)
        gates.append(jnp.where(sel, probs[e] * scale, jnp.float32(0.0)))
    return gates


def _proj_route_body(rs_ref, x_ref, wq_ref, wk_ref, svt_ref, sot_ref,
                     vmat_ref, e12_ref,
                     q_ref, k_ref, vmix_ref, go_ref):
    x = x_ref[...]
    xb = x.astype(jnp.bfloat16)
    s = jnp.float32(_S)
    dn = (((1,), (0,)), ((), ()))
    q = jax.lax.dot_general(xb, wq_ref[...], dn,
                            preferred_element_type=jnp.float32)
    q_ref[...] = (q * s).astype(jnp.bfloat16)
    k = jax.lax.dot_general(xb, wk_ref[...], dn,
                            preferred_element_type=jnp.float32)
    k_ref[...] = (k * s).astype(jnp.bfloat16)

    rs = rs_ref[0, 0]
    lv = jax.lax.dot_general(xb, svt_ref[...], dn,
                             preferred_element_type=jnp.float32)
    gv = _top2_gates(lv, rs)
    lo = jax.lax.dot_general(xb, sot_ref[...], dn,
                             preferred_element_type=jnp.float32)
    go = _top2_gates(lo, rs)
    for e in range(E):
        go_ref[:, e * H:(e + 1) * H] = go[e]

    e12 = e12_ref[...]
    acc = jnp.zeros((TS, HP), jnp.float32)
    for e in range(E):
        av = jax.lax.dot_general(xb, vmat_ref[e], dn,
                                 preferred_element_type=jnp.float32)
        gexp = _expand_gate(gv[e], e12)
        acc = acc + av * gexp
    vmix_ref[...] = acc.astype(jnp.bfloat16)


def _expand_gate(g, e12):
    return jax.lax.dot_general(g.astype(jnp.bfloat16), e12,
                               (((1,), (0,)), ((), ())),
                               preferred_element_type=jnp.float32)


def _attn_body(q_ref, k_ref, v_ref, o_ref):
    qi = pl.program_id(1)
    row = jax.lax.broadcasted_iota(jnp.int32, (TS, TS), 0)
    col = jax.lax.broadcasted_iota(jnp.int32, (TS, TS), 1)
    for j in range(2):
        qv = q_ref[:, j * P:(j + 1) * P]
        parts = []
        for ki in range(NT):
            def _compute(ki=ki, qv=qv, j=j):
                kv = k_ref[ki * TS:(ki + 1) * TS, j * P:(j + 1) * P]
                s = jax.lax.dot_general(qv, kv, (((1,), (1,)), ((), ())),
                                        preferred_element_type=jnp.float32)
                s = jnp.where(ki * TS + col <= qi * TS + row, s,
                              jnp.float32(-1e30))
                m = jnp.max(s, axis=1, keepdims=True)
                p = jnp.exp(s - m)
                l = jnp.sum(p, axis=1, keepdims=True)
                pv = jax.lax.dot_general(
                    p.astype(jnp.bfloat16),
                    v_ref[ki * TS:(ki + 1) * TS, j * P:(j + 1) * P],
                    (((1,), (0,)), ((), ())),
                    preferred_element_type=jnp.float32)
                return m, l, pv

            def _skip():
                return (jnp.full((TS, 1), -1e30, jnp.float32),
                        jnp.zeros((TS, 1), jnp.float32),
                        jnp.zeros((TS, P), jnp.float32))

            if ki == 0:
                parts.append(_compute())
            else:
                parts.append(jax.lax.cond(ki <= qi, _compute, _skip))
        mm = parts[0][0]
        for ki in range(1, NT):
            mm = jnp.maximum(mm, parts[ki][0])
        lsum = jnp.zeros((TS, 1), jnp.float32)
        acc = jnp.zeros((TS, P), jnp.float32)
        for m, l, pv in parts:
            corr = jnp.exp(m - mm)
            lsum = lsum + l * corr
            acc = acc + pv * corr
        o_ref[:, j * P:(j + 1) * P] = (acc / lsum).astype(jnp.bfloat16)


def _oproj_body(res_ref, go_ref, omat_ref, e12_ref, out_ref):
    res = res_ref[...].astype(jnp.float32)
    e12 = e12_ref[...]
    dn = (((1,), (0,)), ((), ()))
    acc = jnp.zeros((TS, D), jnp.float32)
    for e in range(E):
        gexp = _expand_gate(go_ref[:, e * H:(e + 1) * H], e12)
        wres = (res * gexp).astype(jnp.bfloat16)
        acc = acc + jax.lax.dot_general(wres, omat_ref[e], dn,
                                        preferred_element_type=jnp.float32)
    out_ref[...] = acc


@jax.jit
def kernel(x, Wq, Wk, v, o, sel_v, sel_o, route_scale):
    x2 = x[0]
    wqT = Wq.T.astype(jnp.bfloat16)
    wkT = Wk.T.astype(jnp.bfloat16)
    svt = sel_v.reshape(H, E, D).transpose(1, 0, 2).reshape(E * H, D).T
    svt = svt.astype(jnp.bfloat16)
    sot = sel_o.reshape(H, E, D).transpose(1, 0, 2).reshape(E * H, D).T
    sot = sot.astype(jnp.bfloat16)
    vmat = v.astype(jnp.bfloat16).reshape(H, E, D, P)
    vmat = vmat.transpose(1, 2, 0, 3).reshape(E, D, HP)
    omat = o.astype(jnp.bfloat16).reshape(H, E, P, D)
    omat = omat.transpose(1, 0, 2, 3).reshape(E, HP, D)
    rs = route_scale.reshape(1, 1)
    e12 = jnp.repeat(jnp.eye(H, dtype=jnp.bfloat16), P, axis=1)

    def full(shape):
        return pl.BlockSpec(shape, lambda *_: (0,) * len(shape))

    qk, kk, vmixk, gok = pl.pallas_call(
        _proj_route_body,
        grid=(NT,),
        in_specs=[
            pl.BlockSpec(memory_space=pltpu.SMEM),
            pl.BlockSpec((TS, D), lambda i: (i, 0)),
            full((D, HP)),
            full((D, HP)),
            full((D, E * H)),
            full((D, E * H)),
            full((E, D, HP)),
            full((H, HP)),
        ],
        out_specs=[
            pl.BlockSpec((TS, HP), lambda i: (i, 0)),
            pl.BlockSpec((TS, HP), lambda i: (i, 0)),
            pl.BlockSpec((TS, HP), lambda i: (i, 0)),
            pl.BlockSpec((TS, E * H), lambda i: (i, 0)),
        ],
        out_shape=[
            jax.ShapeDtypeStruct((S, HP), jnp.bfloat16),
            jax.ShapeDtypeStruct((S, HP), jnp.bfloat16),
            jax.ShapeDtypeStruct((S, HP), jnp.bfloat16),
            jax.ShapeDtypeStruct((S, E * H), jnp.float32),
        ],
        compiler_params=pltpu.CompilerParams(
            dimension_semantics=("parallel",)),
    )(rs, x2, wqT, wkT, svt, sot, vmat, e12)

    res = pl.pallas_call(
        _attn_body,
        grid=(H // 2, NT),
        in_specs=[
            pl.BlockSpec((TS, 2 * P), lambda h, i: (i, h)),
            pl.BlockSpec((S, 2 * P), lambda h, i: (0, h)),
            pl.BlockSpec((S, 2 * P), lambda h, i: (0, h)),
        ],
        out_specs=pl.BlockSpec((TS, 2 * P), lambda h, i: (i, h)),
        out_shape=jax.ShapeDtypeStruct((S, HP), jnp.bfloat16),
        compiler_params=pltpu.CompilerParams(
            dimension_semantics=("parallel", "parallel")),
    )(qk, kk, vmixk)

    out = pl.pallas_call(
        _oproj_body,
        grid=(NT,),
        in_specs=[
            pl.BlockSpec((TS, HP), lambda i: (i, 0)),
            pl.BlockSpec((TS, E * H), lambda i: (i, 0)),
            full((E, HP, D)),
            full((H, HP)),
        ],
        out_specs=pl.BlockSpec((TS, D), lambda i: (i, 0)),
        out_shape=jax.ShapeDtypeStruct((S, D), jnp.float32),
        compiler_params=pltpu.CompilerParams(
            dimension_semantics=("parallel",)),
    )(res, gok, omat, e12)

    return out.reshape(B, S, D)


# speedup vs baseline: 1.5202x; 1.5202x over previous
import math

import jax
import jax.numpy as jnp
from jax.experimental import pallas as pl
from jax.experimental.pallas import tpu as pltpu

B, S, D = 1, 2048, 768
H, E, TOPK, P = 12, 8, 2, 64
TS = 256
NT = S // TS
HP = H * P

_SCALE = 1.0 / math.sqrt(P)
_S = math.sqrt(_SCALE)

_HI = jax.lax.Precision.HIGHEST


def _top2_gates(logits, rs_over):
    probs = [jax.nn.sigmoid(logits[:, e * H:(e + 1) * H]) for e in range(E)]
    m1 = probs[0]
    for e in range(1, E):
        m1 = jnp.maximum(m1, probs[e])
    i1 = jnp.full(probs[0].shape, E, dtype=jnp.int32)
    for e in range(E - 1, -1, -1):
        i1 = jnp.where(probs[e] == m1, e, i1)
    neg = jnp.float32(-jnp.inf)
    q = [jnp.where(i1 == e, neg, probs[e]) for e in range(E)]
    m2 = q[0]
    for e in range(1, E):
        m2 = jnp.maximum(m2, q[e])
    i2 = jnp.full(probs[0].shape, E, dtype=jnp.int32)
    for e in range(E - 1, -1, -1):
        i2 = jnp.where(q[e] == m2, e, i2)
    denom = jnp.maximum(m1 + m2, jnp.float32(1e-9))
    scale = rs_over / denom
    gates = []
    for e in range(E):
        sel = jnp.logical_or(i1 == e, i2 == e)
        gates.append(jnp.where(sel, probs[e] * scale, jnp.float32(0.0)))
    return gates


def _proj_route_body(rs_ref, x_ref, wq_ref, wk_ref, svt_ref, sot_ref,
                     vmat_ref, e12_ref,
                     q_ref, k_ref, vmix_ref, go_ref):
    x = x_ref[...]
    xb = x.astype(jnp.bfloat16)
    s = jnp.float32(_S)
    dn = (((1,), (0,)), ((), ()))
    q = jax.lax.dot_general(xb, wq_ref[...], dn,
                            preferred_element_type=jnp.float32)
    q_ref[...] = (q * s).astype(jnp.bfloat16)
    k = jax.lax.dot_general(xb, wk_ref[...], dn,
                            preferred_element_type=jnp.float32)
    k_ref[...] = (k * s).astype(jnp.bfloat16)

    rs = rs_ref[0, 0]
    lv = jax.lax.dot_general(xb, svt_ref[...], dn,
                             preferred_element_type=jnp.float32)
    gv = _top2_gates(lv, rs)
    lo = jax.lax.dot_general(xb, sot_ref[...], dn,
                             preferred_element_type=jnp.float32)
    go = _top2_gates(lo, rs)
    for e in range(E):
        go_ref[:, e * H:(e + 1) * H] = go[e]

    e12 = e12_ref[...]
    acc = jnp.zeros((TS, HP), jnp.float32)
    for e in range(E):
        av = jax.lax.dot_general(xb, vmat_ref[e], dn,
                                 preferred_element_type=jnp.float32)
        gexp = _expand_gate(gv[e], e12)
        acc = acc + av * gexp
    vmix_ref[...] = acc.astype(jnp.bfloat16)


def _expand_gate(g, e12):
    return jax.lax.dot_general(g.astype(jnp.bfloat16), e12,
                               (((1,), (0,)), ((), ())),
                               preferred_element_type=jnp.float32)


def _attn_body(q_ref, k_ref, v_ref, o_ref):
    qi = pl.program_id(1)
    row = qi * TS + jax.lax.broadcasted_iota(jnp.int32, (TS, S), 0)
    col = jax.lax.broadcasted_iota(jnp.int32, (TS, S), 1)
    mask = col <= row
    for j in range(2):
        qv = q_ref[:, j * P:(j + 1) * P]
        kv = k_ref[:, j * P:(j + 1) * P]
        s = jax.lax.dot_general(qv, kv, (((1,), (1,)), ((), ())),
                                preferred_element_type=jnp.float32)
        s = jnp.where(mask, s, jnp.float32(-1e30))
        m = jnp.max(s, axis=1, keepdims=True)
        p = jnp.exp(s - m)
        l = jnp.sum(p, axis=1, keepdims=True)
        p = (p / l).astype(jnp.bfloat16)
        o = jax.lax.dot_general(p, v_ref[:, j * P:(j + 1) * P],
                                (((1,), (0,)), ((), ())),
                                preferred_element_type=jnp.float32)
        o_ref[:, j * P:(j + 1) * P] = o.astype(jnp.bfloat16)


def _oproj_body(res_ref, go_ref, omat_ref, e12_ref, out_ref):
    res = res_ref[...].astype(jnp.float32)
    e12 = e12_ref[...]
    dn = (((1,), (0,)), ((), ()))
    acc = jnp.zeros((TS, D), jnp.float32)
    for e in range(E):
        gexp = _expand_gate(go_ref[:, e * H:(e + 1) * H], e12)
        wres = (res * gexp).astype(jnp.bfloat16)
        acc = acc + jax.lax.dot_general(wres, omat_ref[e], dn,
                                        preferred_element_type=jnp.float32)
    out_ref[...] = acc


@jax.jit
def kernel(x, Wq, Wk, v, o, sel_v, sel_o, route_scale):
    x2 = x[0]
    wqT = Wq.T.astype(jnp.bfloat16)
    wkT = Wk.T.astype(jnp.bfloat16)
    svt = sel_v.reshape(H, E, D).transpose(1, 0, 2).reshape(E * H, D).T
    svt = svt.astype(jnp.bfloat16)
    sot = sel_o.reshape(H, E, D).transpose(1, 0, 2).reshape(E * H, D).T
    sot = sot.astype(jnp.bfloat16)
    vmat = v.astype(jnp.bfloat16).reshape(H, E, D, P)
    vmat = vmat.transpose(1, 2, 0, 3).reshape(E, D, HP)
    omat = o.astype(jnp.bfloat16).reshape(H, E, P, D)
    omat = omat.transpose(1, 0, 2, 3).reshape(E, HP, D)
    rs = route_scale.reshape(1, 1)
    e12 = jnp.repeat(jnp.eye(H, dtype=jnp.bfloat16), P, axis=1)

    def full(shape):
        return pl.BlockSpec(shape, lambda *_: (0,) * len(shape))

    qk, kk, vmixk, gok = pl.pallas_call(
        _proj_route_body,
        grid=(NT,),
        in_specs=[
            pl.BlockSpec(memory_space=pltpu.SMEM),
            pl.BlockSpec((TS, D), lambda i: (i, 0)),
            full((D, HP)),
            full((D, HP)),
            full((D, E * H)),
            full((D, E * H)),
            full((E, D, HP)),
            full((H, HP)),
        ],
        out_specs=[
            pl.BlockSpec((TS, HP), lambda i: (i, 0)),
            pl.BlockSpec((TS, HP), lambda i: (i, 0)),
            pl.BlockSpec((TS, HP), lambda i: (i, 0)),
            pl.BlockSpec((TS, E * H), lambda i: (i, 0)),
        ],
        out_shape=[
            jax.ShapeDtypeStruct((S, HP), jnp.bfloat16),
            jax.ShapeDtypeStruct((S, HP), jnp.bfloat16),
            jax.ShapeDtypeStruct((S, HP), jnp.bfloat16),
            jax.ShapeDtypeStruct((S, E * H), jnp.float32),
        ],
        compiler_params=pltpu.CompilerParams(
            dimension_semantics=("parallel",)),
    )(rs, x2, wqT, wkT, svt, sot, vmat, e12)

    res = pl.pallas_call(
        _attn_body,
        grid=(H // 2, NT),
        in_specs=[
            pl.BlockSpec((TS, 2 * P), lambda h, i: (i, h)),
            pl.BlockSpec((S, 2 * P), lambda h, i: (0, h)),
            pl.BlockSpec((S, 2 * P), lambda h, i: (0, h)),
        ],
        out_specs=pl.BlockSpec((TS, 2 * P), lambda h, i: (i, h)),
        out_shape=jax.ShapeDtypeStruct((S, HP), jnp.bfloat16),
        compiler_params=pltpu.CompilerParams(
            dimension_semantics=("parallel", "parallel")),
    )(qk, kk, vmixk)

    out = pl.pallas_call(
        _oproj_body,
        grid=(NT,),
        in_specs=[
            pl.BlockSpec((TS, HP), lambda i: (i, 0)),
            pl.BlockSpec((TS, E * H), lambda i: (i, 0)),
            full((E, HP, D)),
            full((H, HP)),
        ],
        out_specs=pl.BlockSpec((TS, D), lambda i: (i, 0)),
        out_shape=jax.ShapeDtypeStruct((S, D), jnp.float32),
        compiler_params=pltpu.CompilerParams(
            dimension_semantics=("parallel",)),
    )(res, gok, omat, e12)

    return out.reshape(B, S, D)


# speedup vs baseline: 1.6703x; 1.0988x over previous
import math

import jax
import jax.numpy as jnp
from jax.experimental import pallas as pl
from jax.experimental.pallas import tpu as pltpu

B, S, D = 1, 2048, 768
H, E, TOPK, P = 12, 8, 2, 64
TS = 256
NT = S // TS
HP = H * P

_SCALE = 1.0 / math.sqrt(P)
_S = math.sqrt(_SCALE)

_HI = jax.lax.Precision.HIGHEST


def _top2_gates(logits, rs_over):
    probs = [jax.nn.sigmoid(logits[:, e * H:(e + 1) * H]) for e in range(E)]
    m1 = probs[0]
    for e in range(1, E):
        m1 = jnp.maximum(m1, probs[e])
    i1 = jnp.full(probs[0].shape, E, dtype=jnp.int32)
    for e in range(E - 1, -1, -1):
        i1 = jnp.where(probs[e] == m1, e, i1)
    neg = jnp.float32(-jnp.inf)
    q = [jnp.where(i1 == e, neg, probs[e]) for e in range(E)]
    m2 = q[0]
    for e in range(1, E):
        m2 = jnp.maximum(m2, q[e])
    i2 = jnp.full(probs[0].shape, E, dtype=jnp.int32)
    for e in range(E - 1, -1, -1):
        i2 = jnp.where(q[e] == m2, e, i2)
    denom = jnp.maximum(m1 + m2, jnp.float32(1e-9))
    scale = rs_over / denom
    gates = []
    for e in range(E):
        sel = jnp.logical_or(i1 == e, i2 == e)
        gates.append(jnp.where(sel, probs[e] * scale, jnp.float32(0.0)))
    return gates


def _proj_route_body(rs_ref, x_ref, wq_ref, wk_ref, svt_ref, sot_ref,
                     vmat_ref, e12_ref,
                     q_ref, k_ref, vmix_ref, go_ref):
    x = x_ref[...]
    xb = x.astype(jnp.bfloat16)
    s = jnp.float32(_S)
    dn = (((1,), (0,)), ((), ()))
    q = jax.lax.dot_general(xb, wq_ref[...], dn,
                            preferred_element_type=jnp.float32)
    q_ref[...] = (q * s).astype(jnp.bfloat16)
    k = jax.lax.dot_general(xb, wk_ref[...], dn,
                            preferred_element_type=jnp.float32)
    k_ref[...] = (k * s).astype(jnp.bfloat16)

    rs = rs_ref[0, 0]
    lv = jax.lax.dot_general(xb, svt_ref[...], dn,
                             preferred_element_type=jnp.float32)
    gv = _top2_gates(lv, rs)
    lo = jax.lax.dot_general(xb, sot_ref[...], dn,
                             preferred_element_type=jnp.float32)
    go = _top2_gates(lo, rs)
    for e in range(E):
        go_ref[:, e * H:(e + 1) * H] = go[e]

    e12 = e12_ref[...]
    acc = jnp.zeros((TS, HP), jnp.float32)
    for e in range(E):
        av = jax.lax.dot_general(xb, vmat_ref[e], dn,
                                 preferred_element_type=jnp.float32)
        gexp = _expand_gate(gv[e], e12)
        acc = acc + av * gexp
    vmix_ref[...] = acc.astype(jnp.bfloat16)


def _expand_gate(g, e12):
    return jax.lax.dot_general(g.astype(jnp.bfloat16), e12,
                               (((1,), (0,)), ((), ())),
                               preferred_element_type=jnp.float32)


def _make_attn_body(q_start, klen):
    def body(q_ref, k_ref, v_ref, o_ref):
        i = pl.program_id(1)
        row = (q_start + i) * TS + jax.lax.broadcasted_iota(
            jnp.int32, (TS, klen), 0)
        col = jax.lax.broadcasted_iota(jnp.int32, (TS, klen), 1)
        mask = col <= row
        for j in range(2):
            qv = q_ref[:, j * P:(j + 1) * P]
            kv = k_ref[:, j * P:(j + 1) * P]
            s = jax.lax.dot_general(qv, kv, (((1,), (1,)), ((), ())),
                                    preferred_element_type=jnp.float32)
            s = jnp.where(mask, s, jnp.float32(-1e30))
            m = jnp.max(s, axis=1, keepdims=True)
            p = jnp.exp(s - m)
            l = jnp.sum(p, axis=1, keepdims=True)
            p = (p / l).astype(jnp.bfloat16)
            o = jax.lax.dot_general(p, v_ref[:, j * P:(j + 1) * P],
                                    (((1,), (0,)), ((), ())),
                                    preferred_element_type=jnp.float32)
            o_ref[:, j * P:(j + 1) * P] = o.astype(jnp.bfloat16)
    return body


def _oproj_body(res_ref, go_ref, omat_ref, e12_ref, out_ref):
    res = res_ref[...].astype(jnp.float32)
    e12 = e12_ref[...]
    dn = (((1,), (0,)), ((), ()))
    acc = jnp.zeros((TS, D), jnp.float32)
    for e in range(E):
        gexp = _expand_gate(go_ref[:, e * H:(e + 1) * H], e12)
        wres = (res * gexp).astype(jnp.bfloat16)
        acc = acc + jax.lax.dot_general(wres, omat_ref[e], dn,
                                        preferred_element_type=jnp.float32)
    out_ref[...] = acc


@jax.jit
def kernel(x, Wq, Wk, v, o, sel_v, sel_o, route_scale):
    x2 = x[0]
    wqT = Wq.T.astype(jnp.bfloat16)
    wkT = Wk.T.astype(jnp.bfloat16)
    svt = sel_v.reshape(H, E, D).transpose(1, 0, 2).reshape(E * H, D).T
    svt = svt.astype(jnp.bfloat16)
    sot = sel_o.reshape(H, E, D).transpose(1, 0, 2).reshape(E * H, D).T
    sot = sot.astype(jnp.bfloat16)
    vmat = v.astype(jnp.bfloat16).reshape(H, E, D, P)
    vmat = vmat.transpose(1, 2, 0, 3).reshape(E, D, HP)
    omat = o.astype(jnp.bfloat16).reshape(H, E, P, D)
    omat = omat.transpose(1, 0, 2, 3).reshape(E, HP, D)
    rs = route_scale.reshape(1, 1)
    e12 = jnp.repeat(jnp.eye(H, dtype=jnp.bfloat16), P, axis=1)

    def full(shape):
        return pl.BlockSpec(shape, lambda *_: (0,) * len(shape))

    qk, kk, vmixk, gok = pl.pallas_call(
        _proj_route_body,
        grid=(NT,),
        in_specs=[
            pl.BlockSpec(memory_space=pltpu.SMEM),
            pl.BlockSpec((TS, D), lambda i: (i, 0)),
            full((D, HP)),
            full((D, HP)),
            full((D, E * H)),
            full((D, E * H)),
            full((E, D, HP)),
            full((H, HP)),
        ],
        out_specs=[
            pl.BlockSpec((TS, HP), lambda i: (i, 0)),
            pl.BlockSpec((TS, HP), lambda i: (i, 0)),
            pl.BlockSpec((TS, HP), lambda i: (i, 0)),
            pl.BlockSpec((TS, E * H), lambda i: (i, 0)),
        ],
        out_shape=[
            jax.ShapeDtypeStruct((S, HP), jnp.bfloat16),
            jax.ShapeDtypeStruct((S, HP), jnp.bfloat16),
            jax.ShapeDtypeStruct((S, HP), jnp.bfloat16),
            jax.ShapeDtypeStruct((S, E * H), jnp.float32),
        ],
        compiler_params=pltpu.CompilerParams(
            dimension_semantics=("parallel",)),
    )(rs, x2, wqT, wkT, svt, sot, vmat, e12)

    parts = []
    for ci in range(4):
        q_start, klen = 2 * ci, (2 * ci + 2) * TS
        parts.append(pl.pallas_call(
            _make_attn_body(q_start, klen),
            grid=(H // 2, 2),
            in_specs=[
                pl.BlockSpec((TS, 2 * P),
                             lambda h, i, qs=q_start: (qs + i, h)),
                pl.BlockSpec((klen, 2 * P), lambda h, i: (0, h)),
                pl.BlockSpec((klen, 2 * P), lambda h, i: (0, h)),
            ],
            out_specs=pl.BlockSpec((TS, 2 * P), lambda h, i: (i, h)),
            out_shape=jax.ShapeDtypeStruct((2 * TS, HP), jnp.bfloat16),
            compiler_params=pltpu.CompilerParams(
                dimension_semantics=("parallel", "parallel")),
        )(qk, kk, vmixk))
    res = jnp.concatenate(parts, axis=0)

    out = pl.pallas_call(
        _oproj_body,
        grid=(NT,),
        in_specs=[
            pl.BlockSpec((TS, HP), lambda i: (i, 0)),
            pl.BlockSpec((TS, E * H), lambda i: (i, 0)),
            full((E, HP, D)),
            full((H, HP)),
        ],
        out_specs=pl.BlockSpec((TS, D), lambda i: (i, 0)),
        out_shape=jax.ShapeDtypeStruct((S, D), jnp.float32),
        compiler_params=pltpu.CompilerParams(
            dimension_semantics=("parallel",)),
    )(res, gok, omat, e12)

    return out.reshape(B, S, D)


# speedup vs baseline: 1.6862x; 1.0095x over previous
import math

import jax
import jax.numpy as jnp
from jax.experimental import pallas as pl
from jax.experimental.pallas import tpu as pltpu

B, S, D = 1, 2048, 768
H, E, TOPK, P = 12, 8, 2, 64
TS = 256
NT = S // TS
HP = H * P

_SCALE = 1.0 / math.sqrt(P)
_S = math.sqrt(_SCALE)

_HI = jax.lax.Precision.HIGHEST


def _prep_body(v_ref, o_ref, vmat_ref, omat_ref):
    for h in range(H):
        vmat_ref[0, :, h * P:(h + 1) * P] = v_ref[h, 0].astype(jnp.bfloat16)
        omat_ref[0, h * P:(h + 1) * P, :] = o_ref[h, 0].astype(jnp.bfloat16)


def _top2_gates(logits, rs_over):
    probs = [jax.nn.sigmoid(logits[:, e * H:(e + 1) * H]) for e in range(E)]
    m1 = probs[0]
    for e in range(1, E):
        m1 = jnp.maximum(m1, probs[e])
    i1 = jnp.full(probs[0].shape, E, dtype=jnp.int32)
    for e in range(E - 1, -1, -1):
        i1 = jnp.where(probs[e] == m1, e, i1)
    neg = jnp.float32(-jnp.inf)
    q = [jnp.where(i1 == e, neg, probs[e]) for e in range(E)]
    m2 = q[0]
    for e in range(1, E):
        m2 = jnp.maximum(m2, q[e])
    i2 = jnp.full(probs[0].shape, E, dtype=jnp.int32)
    for e in range(E - 1, -1, -1):
        i2 = jnp.where(q[e] == m2, e, i2)
    denom = jnp.maximum(m1 + m2, jnp.float32(1e-9))
    scale = rs_over / denom
    gates = []
    for e in range(E):
        sel = jnp.logical_or(i1 == e, i2 == e)
        gates.append(jnp.where(sel, probs[e] * scale, jnp.float32(0.0)))
    return gates


def _proj_route_body(rs_ref, x_ref, wq_ref, wk_ref, svt_ref, sot_ref,
                     vmat_ref, e12_ref,
                     q_ref, k_ref, vmix_ref, go_ref):
    xb = x_ref[...]
    s = jnp.float32(_S)
    dn = (((1,), (0,)), ((), ()))
    dnt = (((1,), (1,)), ((), ()))
    q = jax.lax.dot_general(xb, wq_ref[...], dnt,
                            preferred_element_type=jnp.float32)
    q_ref[...] = (q * s).astype(jnp.bfloat16)
    k = jax.lax.dot_general(xb, wk_ref[...], dnt,
                            preferred_element_type=jnp.float32)
    k_ref[...] = (k * s).astype(jnp.bfloat16)

    rs = rs_ref[0, 0]
    lv = jax.lax.dot_general(xb, svt_ref[...], dnt,
                             preferred_element_type=jnp.float32)
    gv = _top2_gates(lv, rs)
    lo = jax.lax.dot_general(xb, sot_ref[...], dnt,
                             preferred_element_type=jnp.float32)
    go = _top2_gates(lo, rs)
    for e in range(E):
        go_ref[:, e * H:(e + 1) * H] = go[e]

    e12 = e12_ref[...]
    acc = jnp.zeros((TS, HP), jnp.float32)
    for e in range(E):
        av = jax.lax.dot_general(xb, vmat_ref[e], dn,
                                 preferred_element_type=jnp.float32)
        gexp = _expand_gate(gv[e], e12)
        acc = acc + av * gexp
    vmix_ref[...] = acc.astype(jnp.bfloat16)


def _expand_gate(g, e12):
    return jax.lax.dot_general(g.astype(jnp.bfloat16), e12,
                               (((1,), (0,)), ((), ())),
                               preferred_element_type=jnp.float32)


def _make_attn_body(q_start, klen):
    def body(q_ref, k_ref, v_ref, o_ref):
        i = pl.program_id(1)
        row = (q_start + i) * TS + jax.lax.broadcasted_iota(
            jnp.int32, (TS, klen), 0)
        col = jax.lax.broadcasted_iota(jnp.int32, (TS, klen), 1)
        mask = col <= row
        for j in range(2):
            qv = q_ref[:, j * P:(j + 1) * P]
            kv = k_ref[:, j * P:(j + 1) * P]
            s = jax.lax.dot_general(qv, kv, (((1,), (1,)), ((), ())),
                                    preferred_element_type=jnp.float32)
            s = jnp.where(mask, s, jnp.float32(-1e30))
            m = jnp.max(s, axis=1, keepdims=True)
            p = jnp.exp(s - m)
            l = jnp.sum(p, axis=1, keepdims=True)
            p = (p / l).astype(jnp.bfloat16)
            o = jax.lax.dot_general(p, v_ref[:, j * P:(j + 1) * P],
                                    (((1,), (0,)), ((), ())),
                                    preferred_element_type=jnp.float32)
            o_ref[:, j * P:(j + 1) * P] = o.astype(jnp.bfloat16)
    return body


def _oproj_body(res_ref, go_ref, omat_ref, e12_ref, out_ref):
    res = res_ref[...].astype(jnp.float32)
    e12 = e12_ref[...]
    dn = (((1,), (0,)), ((), ()))
    acc = jnp.zeros((TS, D), jnp.float32)
    for e in range(E):
        gexp = _expand_gate(go_ref[:, e * H:(e + 1) * H], e12)
        wres = (res * gexp).astype(jnp.bfloat16)
        acc = acc + jax.lax.dot_general(wres, omat_ref[e], dn,
                                        preferred_element_type=jnp.float32)
    out_ref[...] = acc


@jax.jit
def kernel(x, Wq, Wk, v, o, sel_v, sel_o, route_scale):
    xbf = x[0].astype(jnp.bfloat16)
    wqb = Wq.astype(jnp.bfloat16)
    wkb = Wk.astype(jnp.bfloat16)
    svt = sel_v.reshape(H, E, D).transpose(1, 0, 2).reshape(E * H, D)
    svt = svt.astype(jnp.bfloat16)
    sot = sel_o.reshape(H, E, D).transpose(1, 0, 2).reshape(E * H, D)
    sot = sot.astype(jnp.bfloat16)
    rs = route_scale.reshape(1, 1)
    e12 = jnp.repeat(jnp.eye(H, dtype=jnp.bfloat16), P, axis=1)

    def full(shape):
        return pl.BlockSpec(shape, lambda *_: (0,) * len(shape))

    vmat, omat = pl.pallas_call(
        _prep_body,
        grid=(E,),
        in_specs=[
            pl.BlockSpec((H, 1, D, P), lambda e: (0, e, 0, 0)),
            pl.BlockSpec((H, 1, P, D), lambda e: (0, e, 0, 0)),
        ],
        out_specs=[
            pl.BlockSpec((1, D, HP), lambda e: (e, 0, 0)),
            pl.BlockSpec((1, HP, D), lambda e: (e, 0, 0)),
        ],
        out_shape=[
            jax.ShapeDtypeStruct((E, D, HP), jnp.bfloat16),
            jax.ShapeDtypeStruct((E, HP, D), jnp.bfloat16),
        ],
        compiler_params=pltpu.CompilerParams(
            dimension_semantics=("parallel",)),
    )(v.reshape(H, E, D, P), o.reshape(H, E, P, D))

    qk, kk, vmixk, gok = pl.pallas_call(
        _proj_route_body,
        grid=(NT,),
        in_specs=[
            pl.BlockSpec(memory_space=pltpu.SMEM),
            pl.BlockSpec((TS, D), lambda i: (i, 0)),
            full((HP, D)),
            full((HP, D)),
            full((E * H, D)),
            full((E * H, D)),
            full((E, D, HP)),
            full((H, HP)),
        ],
        out_specs=[
            pl.BlockSpec((TS, HP), lambda i: (i, 0)),
            pl.BlockSpec((TS, HP), lambda i: (i, 0)),
            pl.BlockSpec((TS, HP), lambda i: (i, 0)),
            pl.BlockSpec((TS, E * H), lambda i: (i, 0)),
        ],
        out_shape=[
            jax.ShapeDtypeStruct((S, HP), jnp.bfloat16),
            jax.ShapeDtypeStruct((S, HP), jnp.bfloat16),
            jax.ShapeDtypeStruct((S, HP), jnp.bfloat16),
            jax.ShapeDtypeStruct((S, E * H), jnp.float32),
        ],
        compiler_params=pltpu.CompilerParams(
            dimension_semantics=("parallel",)),
    )(rs, xbf, wqb, wkb, svt, sot, vmat, e12)

    parts = []
    for ci in range(4):
        q_start, klen = 2 * ci, (2 * ci + 2) * TS
        parts.append(pl.pallas_call(
            _make_attn_body(q_start, klen),
            grid=(H // 2, 2),
            in_specs=[
                pl.BlockSpec((TS, 2 * P),
                             lambda h, i, qs=q_start: (qs + i, h)),
                pl.BlockSpec((klen, 2 * P), lambda h, i: (0, h)),
                pl.BlockSpec((klen, 2 * P), lambda h, i: (0, h)),
            ],
            out_specs=pl.BlockSpec((TS, 2 * P), lambda h, i: (i, h)),
            out_shape=jax.ShapeDtypeStruct((2 * TS, HP), jnp.bfloat16),
            compiler_params=pltpu.CompilerParams(
                dimension_semantics=("parallel", "parallel")),
        )(qk, kk, vmixk))
    res = jnp.concatenate(parts, axis=0)

    out = pl.pallas_call(
        _oproj_body,
        grid=(NT,),
        in_specs=[
            pl.BlockSpec((TS, HP), lambda i: (i, 0)),
            pl.BlockSpec((TS, E * H), lambda i: (i, 0)),
            full((E, HP, D)),
            full((H, HP)),
        ],
        out_specs=pl.BlockSpec((TS, D), lambda i: (i, 0)),
        out_shape=jax.ShapeDtypeStruct((S, D), jnp.float32),
        compiler_params=pltpu.CompilerParams(
            dimension_semantics=("parallel",)),
    )(res, gok, omat, e12)

    return out.reshape(B, S, D)


# speedup vs baseline: 1.7251x; 1.0231x over previous
import math

import jax
import jax.numpy as jnp
from jax.experimental import pallas as pl
from jax.experimental.pallas import tpu as pltpu

B, S, D = 1, 2048, 768
H, E, TOPK, P = 12, 8, 2, 64
TS = 256
NT = S // TS
HP = H * P

_SCALE = 1.0 / math.sqrt(P)
_S = math.sqrt(_SCALE)

_HI = jax.lax.Precision.HIGHEST


def _prep_body(v_ref, o_ref, vmat_ref, omat_ref):
    for h in range(H):
        vmat_ref[0, :, h * P:(h + 1) * P] = v_ref[h, 0].astype(jnp.bfloat16)
        omat_ref[0, h * P:(h + 1) * P, :] = o_ref[h, 0].astype(jnp.bfloat16)


def _top2_gates(logits, rs_over):
    probs = [jax.nn.sigmoid(logits[:, e * H:(e + 1) * H]) for e in range(E)]
    m1 = probs[0]
    for e in range(1, E):
        m1 = jnp.maximum(m1, probs[e])
    i1 = jnp.full(probs[0].shape, E, dtype=jnp.int32)
    for e in range(E - 1, -1, -1):
        i1 = jnp.where(probs[e] == m1, e, i1)
    neg = jnp.float32(-jnp.inf)
    q = [jnp.where(i1 == e, neg, probs[e]) for e in range(E)]
    m2 = q[0]
    for e in range(1, E):
        m2 = jnp.maximum(m2, q[e])
    i2 = jnp.full(probs[0].shape, E, dtype=jnp.int32)
    for e in range(E - 1, -1, -1):
        i2 = jnp.where(q[e] == m2, e, i2)
    denom = jnp.maximum(m1 + m2, jnp.float32(1e-9))
    scale = rs_over / denom
    gates = []
    for e in range(E):
        sel = jnp.logical_or(i1 == e, i2 == e)
        gates.append(jnp.where(sel, probs[e] * scale, jnp.float32(0.0)))
    return gates


def _proj_route_body(rs_ref, x_ref, wq_ref, wk_ref, svt_ref, sot_ref,
                     vmat_ref, e12_ref,
                     q_ref, k_ref, vmix_ref, go_ref):
    xb = x_ref[...]
    s = jnp.float32(_S)
    dn = (((1,), (0,)), ((), ()))
    dnt = (((1,), (1,)), ((), ()))
    q = jax.lax.dot_general(xb, wq_ref[...], dnt,
                            preferred_element_type=jnp.float32)
    q_ref[...] = (q * s).astype(jnp.bfloat16)
    k = jax.lax.dot_general(xb, wk_ref[...], dnt,
                            preferred_element_type=jnp.float32)
    k_ref[...] = (k * s).astype(jnp.bfloat16)

    rs = rs_ref[0, 0]
    lv = jax.lax.dot_general(xb, svt_ref[...], dnt,
                             preferred_element_type=jnp.float32)
    gv = _top2_gates(lv, rs)
    lo = jax.lax.dot_general(xb, sot_ref[...], dnt,
                             preferred_element_type=jnp.float32)
    go = _top2_gates(lo, rs)
    for e in range(E):
        go_ref[:, e * H:(e + 1) * H] = go[e]

    e12 = e12_ref[...]
    acc = jnp.zeros((TS, HP), jnp.float32)
    for e in range(E):
        av = jax.lax.dot_general(xb, vmat_ref[e], dn,
                                 preferred_element_type=jnp.float32)
        gexp = _expand_gate(gv[e], e12)
        acc = acc + av * gexp
    vmix_ref[...] = acc.astype(jnp.bfloat16)


def _expand_gate(g, e12):
    return jax.lax.dot_general(g.astype(jnp.bfloat16), e12,
                               (((1,), (0,)), ((), ())),
                               preferred_element_type=jnp.float32)


def _make_attn_body(q_start, klen):
    def body(q_ref, k_ref, v_ref, o_ref):
        i = pl.program_id(1)
        row = (q_start + i) * TS + jax.lax.broadcasted_iota(
            jnp.int32, (TS, klen), 0)
        col = jax.lax.broadcasted_iota(jnp.int32, (TS, klen), 1)
        mask = col <= row
        for j in range(4):
            qv = q_ref[:, j * P:(j + 1) * P]
            kv = k_ref[:, j * P:(j + 1) * P]
            s = jax.lax.dot_general(qv, kv, (((1,), (1,)), ((), ())),
                                    preferred_element_type=jnp.float32)
            s = jnp.where(mask, s, jnp.float32(-1e30))
            m = jnp.max(s, axis=1, keepdims=True)
            p = jnp.exp(s - m)
            l = jnp.sum(p, axis=1, keepdims=True)
            p = (p / l).astype(jnp.bfloat16)
            o = jax.lax.dot_general(p, v_ref[:, j * P:(j + 1) * P],
                                    (((1,), (0,)), ((), ())),
                                    preferred_element_type=jnp.float32)
            o_ref[:, j * P:(j + 1) * P] = o.astype(jnp.bfloat16)
    return body


def _oproj_body(res_ref, go_ref, omat_ref, e12_ref, out_ref):
    res = res_ref[...].astype(jnp.float32)
    e12 = e12_ref[...]
    dn = (((1,), (0,)), ((), ()))
    acc = jnp.zeros((TS, D), jnp.float32)
    for e in range(E):
        gexp = _expand_gate(go_ref[:, e * H:(e + 1) * H], e12)
        wres = (res * gexp).astype(jnp.bfloat16)
        acc = acc + jax.lax.dot_general(wres, omat_ref[e], dn,
                                        preferred_element_type=jnp.float32)
    out_ref[...] = acc


@jax.jit
def kernel(x, Wq, Wk, v, o, sel_v, sel_o, route_scale):
    xbf = x[0].astype(jnp.bfloat16)
    wqb = Wq.astype(jnp.bfloat16)
    wkb = Wk.astype(jnp.bfloat16)
    svt = sel_v.reshape(H, E, D).transpose(1, 0, 2).reshape(E * H, D)
    svt = svt.astype(jnp.bfloat16)
    sot = sel_o.reshape(H, E, D).transpose(1, 0, 2).reshape(E * H, D)
    sot = sot.astype(jnp.bfloat16)
    rs = route_scale.reshape(1, 1)
    e12 = jnp.repeat(jnp.eye(H, dtype=jnp.bfloat16), P, axis=1)

    def full(shape):
        return pl.BlockSpec(shape, lambda *_: (0,) * len(shape))

    vmat, omat = pl.pallas_call(
        _prep_body,
        grid=(E,),
        in_specs=[
            pl.BlockSpec((H, 1, D, P), lambda e: (0, e, 0, 0)),
            pl.BlockSpec((H, 1, P, D), lambda e: (0, e, 0, 0)),
        ],
        out_specs=[
            pl.BlockSpec((1, D, HP), lambda e: (e, 0, 0)),
            pl.BlockSpec((1, HP, D), lambda e: (e, 0, 0)),
        ],
        out_shape=[
            jax.ShapeDtypeStruct((E, D, HP), jnp.bfloat16),
            jax.ShapeDtypeStruct((E, HP, D), jnp.bfloat16),
        ],
        compiler_params=pltpu.CompilerParams(
            dimension_semantics=("parallel",)),
    )(v.reshape(H, E, D, P), o.reshape(H, E, P, D))

    qk, kk, vmixk, gok = pl.pallas_call(
        _proj_route_body,
        grid=(NT,),
        in_specs=[
            pl.BlockSpec(memory_space=pltpu.SMEM),
            pl.BlockSpec((TS, D), lambda i: (i, 0)),
            full((HP, D)),
            full((HP, D)),
            full((E * H, D)),
            full((E * H, D)),
            full((E, D, HP)),
            full((H, HP)),
        ],
        out_specs=[
            pl.BlockSpec((TS, HP), lambda i: (i, 0)),
            pl.BlockSpec((TS, HP), lambda i: (i, 0)),
            pl.BlockSpec((TS, HP), lambda i: (i, 0)),
            pl.BlockSpec((TS, E * H), lambda i: (i, 0)),
        ],
        out_shape=[
            jax.ShapeDtypeStruct((S, HP), jnp.bfloat16),
            jax.ShapeDtypeStruct((S, HP), jnp.bfloat16),
            jax.ShapeDtypeStruct((S, HP), jnp.bfloat16),
            jax.ShapeDtypeStruct((S, E * H), jnp.float32),
        ],
        compiler_params=pltpu.CompilerParams(
            dimension_semantics=("parallel",)),
    )(rs, xbf, wqb, wkb, svt, sot, vmat, e12)

    parts = []
    for ci in range(4):
        q_start, klen = 2 * ci, (2 * ci + 2) * TS
        parts.append(pl.pallas_call(
            _make_attn_body(q_start, klen),
            grid=(H // 4, 2),
            in_specs=[
                pl.BlockSpec((TS, 4 * P),
                             lambda h, i, qs=q_start: (qs + i, h)),
                pl.BlockSpec((klen, 4 * P), lambda h, i: (0, h)),
                pl.BlockSpec((klen, 4 * P), lambda h, i: (0, h)),
            ],
            out_specs=pl.BlockSpec((TS, 4 * P), lambda h, i: (i, h)),
            out_shape=jax.ShapeDtypeStruct((2 * TS, HP), jnp.bfloat16),
            compiler_params=pltpu.CompilerParams(
                dimension_semantics=("parallel", "parallel")),
        )(qk, kk, vmixk))
    res = jnp.concatenate(parts, axis=0)

    out = pl.pallas_call(
        _oproj_body,
        grid=(NT,),
        in_specs=[
            pl.BlockSpec((TS, HP), lambda i: (i, 0)),
            pl.BlockSpec((TS, E * H), lambda i: (i, 0)),
            full((E, HP, D)),
            full((H, HP)),
        ],
        out_specs=pl.BlockSpec((TS, D), lambda i: (i, 0)),
        out_shape=jax.ShapeDtypeStruct((S, D), jnp.float32),
        compiler_params=pltpu.CompilerParams(
            dimension_semantics=("parallel",)),
    )(res, gok, omat, e12)

    return out.reshape(B, S, D)
